# Initial kernel scaffold; baseline (speedup 1.0000x reference)
import functools

import jax
import jax.numpy as jnp
from jax import lax
from jax.experimental import pallas as pl
from jax.experimental.pallas import tpu as pltpu
from jax.experimental.pallas import tpu_sc as plsc

VOCAB = 1000000
EMBED_DIM = 64
BATCH = 16384
HIST_LEN = 50

_B = BATCH * HIST_LEN
_NW = 32
_BPW = _B // _NW
_CHUNK = 128
_GPS = 4
_STEP = _CHUNK * _GPS
_NSTEPS = _BPW // _STEP
_ROWS_PER_W = _BPW // _CHUNK

_mesh = plsc.VectorSubcoreMesh(core_axis_name="c", subcore_axis_name="s")


@functools.partial(
    pl.kernel,
    mesh=_mesh,
    out_type=jax.ShapeDtypeStruct((_B, EMBED_DIM), jnp.float32),
    scratch_types=[
        pltpu.VMEM((_ROWS_PER_W, _CHUNK), jnp.int32),
        pltpu.VMEM((_STEP, EMBED_DIM), jnp.float32),
        pltpu.SemaphoreType.DMA,
    ],
)
def _gather_kernel(table_hbm, idx_hbm, out_hbm, idx_v, rows_v, sem):
    wid = lax.axis_index("s") * 2 + lax.axis_index("c")
    base = wid * _BPW
    pltpu.sync_copy(idx_hbm.at[pl.ds(wid * _ROWS_PER_W, _ROWS_PER_W)], idx_v)

    def step(g, carry):
        copies = []
        for i in range(_GPS):
            copies.append(
                pltpu.async_copy(
                    table_hbm.at[idx_v.at[g * _GPS + i]],
                    rows_v.at[pl.ds(i * _CHUNK, _CHUNK)],
                    sem,
                )
            )
        for c in copies:
            c.wait()
        pltpu.sync_copy(rows_v, out_hbm.at[pl.ds(base + g * _STEP, _STEP)])
        return carry

    lax.fori_loop(0, _NSTEPS, step, 0)


def kernel(item_embeddings, batch_data):
    idx = batch_data.reshape(-1).astype(jnp.int32)
    idx2d = idx.reshape(_B // _CHUNK, _CHUNK)
    out = _gather_kernel(item_embeddings, idx2d)
    return out.reshape(BATCH, HIST_LEN, EMBED_DIM)


# speedup vs baseline: 1.8318x; 1.8318x over previous
import functools

import jax
import jax.numpy as jnp
from jax import lax
from jax.experimental import pallas as pl
from jax.experimental.pallas import tpu as pltpu
from jax.experimental.pallas import tpu_sc as plsc

VOCAB = 1000000
EMBED_DIM = 64
BATCH = 16384
HIST_LEN = 50

_B = BATCH * HIST_LEN
_NW = 32
_BPW = _B // _NW
_CHUNK = 128
_GPS = 4
_STEP = _CHUNK * _GPS
_NSTEPS = _BPW // _STEP
_ROWS_PER_W = _BPW // _CHUNK

_mesh = plsc.VectorSubcoreMesh(core_axis_name="c", subcore_axis_name="s")


@functools.partial(
    pl.kernel,
    mesh=_mesh,
    out_type=jax.ShapeDtypeStruct((_B, EMBED_DIM), jnp.float32),
    scratch_types=[
        pltpu.VMEM((_ROWS_PER_W, _CHUNK), jnp.int32),
        pltpu.VMEM((_STEP, EMBED_DIM), jnp.float32),
        pltpu.SemaphoreType.DMA,
    ],
    compiler_params=pltpu.CompilerParams(use_tc_tiling_on_sc=False),
)
def _gather_kernel(table_hbm, idx_hbm, out_hbm, idx_v, rows_v, sem):
    wid = lax.axis_index("s") * 2 + lax.axis_index("c")
    base = wid * _BPW
    pltpu.sync_copy(idx_hbm.at[pl.ds(wid * _ROWS_PER_W, _ROWS_PER_W)], idx_v)

    def step(g, carry):
        copies = []
        for i in range(_GPS):
            copies.append(
                pltpu.async_copy(
                    table_hbm.at[idx_v.at[g * _GPS + i]],
                    rows_v.at[pl.ds(i * _CHUNK, _CHUNK)],
                    sem,
                )
            )
        for c in copies:
            c.wait()
        pltpu.sync_copy(rows_v, out_hbm.at[pl.ds(base + g * _STEP, _STEP)])
        return carry

    lax.fori_loop(0, _NSTEPS, step, 0)


def kernel(item_embeddings, batch_data):
    idx = batch_data.reshape(-1).astype(jnp.int32)
    idx2d = idx.reshape(_B // _CHUNK, _CHUNK)
    out = _gather_kernel(item_embeddings, idx2d)
    return out.reshape(BATCH, HIST_LEN, EMBED_DIM)


# speedup vs baseline: 1.8753x; 1.0237x over previous
import functools

import jax
import jax.numpy as jnp
from jax import lax
from jax.experimental import pallas as pl
from jax.experimental.pallas import tpu as pltpu
from jax.experimental.pallas import tpu_sc as plsc

VOCAB = 1000000
EMBED_DIM = 64
BATCH = 16384
HIST_LEN = 50

_B = BATCH * HIST_LEN
_NW = 32
_BPW = _B // _NW
_CHUNK = 128
_GPS = 5
_STEP = _CHUNK * _GPS
_NSTEPS = _BPW // _STEP
_NBUF = 2
_NOUTER = _NSTEPS // _NBUF
_ROWS_PER_W = _BPW // _CHUNK

_mesh = plsc.VectorSubcoreMesh(core_axis_name="c", subcore_axis_name="s")


@functools.partial(
    pl.kernel,
    mesh=_mesh,
    out_type=jax.ShapeDtypeStruct((_B, EMBED_DIM), jnp.float32),
    scratch_types=[
        pltpu.VMEM((_ROWS_PER_W, _CHUNK), jnp.int32),
        [pltpu.VMEM((_STEP, EMBED_DIM), jnp.float32) for _ in range(_NBUF)],
        [pltpu.SemaphoreType.DMA for _ in range(_NBUF)],
    ],
    compiler_params=pltpu.CompilerParams(use_tc_tiling_on_sc=False),
)
def _gather_kernel(table_hbm, idx_hbm, out_hbm, idx_v, rows_bufs, sems):
    wid = lax.axis_index("s") * 2 + lax.axis_index("c")
    base = wid * _BPW
    pltpu.sync_copy(idx_hbm.at[pl.ds(wid * _ROWS_PER_W, _ROWS_PER_W)], idx_v)

    def fire(step, b):
        for i in range(_GPS):
            pltpu.async_copy(
                table_hbm.at[idx_v.at[step * _GPS + i]],
                rows_bufs[b].at[pl.ds(i * _CHUNK, _CHUNK)],
                sems[b],
            )

    def drain_and_store(step, b):
        pltpu.make_async_copy(
            table_hbm.at[pl.ds(0, _STEP)], rows_bufs[b], sems[b]
        ).wait()
        pltpu.sync_copy(rows_bufs[b], out_hbm.at[pl.ds(base + step * _STEP, _STEP)])

    for b in range(_NBUF):
        fire(b, b)

    def outer(t, carry):
        for b in range(_NBUF):
            step = t * _NBUF + b
            drain_and_store(step, b)
            fire(step + _NBUF, b)
        return carry

    lax.fori_loop(0, _NOUTER - 1, outer, 0)

    for b in range(_NBUF):
        drain_and_store((_NOUTER - 1) * _NBUF + b, b)


def kernel(item_embeddings, batch_data):
    idx = batch_data.reshape(-1).astype(jnp.int32)
    idx2d = idx.reshape(_B // _CHUNK, _CHUNK)
    out = _gather_kernel(item_embeddings, idx2d)
    return out.reshape(BATCH, HIST_LEN, EMBED_DIM)
